# baseline (speedup 1.0000x reference)
import functools

import jax
import jax.numpy as jnp
from jax import lax
from jax.experimental import pallas as pl
from jax.experimental.pallas import tpu as pltpu
from jax.experimental.pallas import tpu_sc as plsc

DURATION = 100.0
NWORKERS = 32


def _vgather(v, idx):
    return jax.lax.gather(
        v,
        idx[:, None],
        jax.lax.GatherDimensionNumbers(
            offset_dims=(), collapsed_slice_dims=(0,), start_index_map=(0,)
        ),
        (1,),
        mode=jax.lax.GatherScatterMode.PROMISE_IN_BOUNDS,
    )


def _sc_hist_body(off_hbm, out_hbm, buf, cnt_buf):
    nclass = 10
    nwords = off_hbm.shape[0] // NWORKERS
    nper = nwords // 16
    wid = lax.axis_index("s") * 2 + lax.axis_index("c")
    base = wid * nwords
    pltpu.sync_copy(off_hbm.at[pl.ds(base, nwords)], buf)

    lanes = lax.broadcasted_iota(jnp.int32, (16,), 0)
    real = lanes < nclass
    neg = jnp.full((16,), -3.4e38, jnp.float32)

    def neuron(i, cnt):
        v = jnp.where(real, buf[pl.ds(i * 16, 16)], neg)
        m = v
        for sh in (1, 2, 4, 8):
            m = jnp.maximum(m, _vgather(m, lanes ^ sh))
        a = jnp.where(v == m, lanes, 16)
        for sh in (1, 2, 4, 8):
            a = jnp.minimum(a, _vgather(a, lanes ^ sh))
        return cnt + jnp.where(lanes == a, 1.0, 0.0)

    cnt = lax.fori_loop(0, nper, neuron, jnp.zeros((16,), jnp.float32))
    cnt_buf[...] = cnt
    pltpu.sync_copy(cnt_buf, out_hbm.at[wid])


def _sc_histogram(offs_flat):
    nwords = offs_flat.shape[0] // NWORKERS
    mesh = plsc.VectorSubcoreMesh(
        core_axis_name="c", subcore_axis_name="s", num_cores=2
    )
    k = functools.partial(
        pl.kernel,
        mesh=mesh,
        out_type=jax.ShapeDtypeStruct((NWORKERS, 16), jnp.float32),
        scratch_types=[
            pltpu.VMEM((nwords,), jnp.float32),
            pltpu.VMEM((16,), jnp.float32),
        ],
    )(_sc_hist_body)
    return k(offs_flat)


def _tc_stream_body(x_ref, off_ref, acc_ref):
    i = pl.program_id(0)
    nclass = off_ref.shape[1]

    off = off_ref[...]
    norms = jnp.sum(jnp.abs(off), axis=1, keepdims=True)
    prop = off / jnp.maximum(norms, 1e-12)
    maxv = jnp.max(prop, axis=1, keepdims=True)
    iota = jax.lax.broadcasted_iota(jnp.int32, prop.shape, 1)
    amax = jnp.min(jnp.where(prop == maxv, iota, nclass), axis=1, keepdims=True)
    assoc = jnp.where(iota == amax, prop, 0.0)

    @pl.when(i == 0)
    def _init():
        acc_ref[...] = jnp.zeros_like(acc_ref)

    x = ((DURATION - x_ref[...]) * (1.0 / DURATION)).astype(jnp.bfloat16)
    acc_ref[...] += jnp.dot(
        x, assoc.astype(jnp.bfloat16), preferred_element_type=jnp.float32
    )


def _tc_stream(inputs, offsets):
    batch, nneuron = inputs.shape
    nclass = offsets.shape[1]
    blk_n = 4096
    grid = nneuron // blk_n
    return pl.pallas_call(
        _tc_stream_body,
        grid=(grid,),
        in_specs=[
            pl.BlockSpec((batch, blk_n), lambda i: (0, i)),
            pl.BlockSpec((blk_n, nclass), lambda i: (i, 0)),
        ],
        out_specs=pl.BlockSpec((batch, nclass), lambda i: (0, 0)),
        out_shape=jax.ShapeDtypeStruct((batch, nclass), jnp.float32),
        compiler_params=pltpu.CompilerParams(
            dimension_semantics=("arbitrary",),
        ),
    )(inputs, offsets)


def _tc_finalize_body(acc_ref, parts_ref, out_ref):
    nclass = acc_ref.shape[1]
    occ = jnp.sum(parts_ref[...], axis=0)[:nclass]
    out_ref[...] = acc_ref[...] / jnp.maximum(occ, 1.0)[None, :]


def _tc_finalize(acc, parts):
    return pl.pallas_call(
        _tc_finalize_body,
        out_shape=jax.ShapeDtypeStruct(acc.shape, jnp.float32),
    )(acc, parts)


def kernel(inputs, offsets):
    offs16 = jnp.pad(offsets, ((0, 0), (0, 16 - offsets.shape[1])))
    occ_parts = _sc_histogram(offs16.reshape(-1))
    acc = _tc_stream(inputs, offsets)
    return _tc_finalize(acc, occ_parts)


# speedup vs baseline: 1.4898x; 1.4898x over previous
import functools

import jax
import jax.numpy as jnp
from jax import lax
from jax.experimental import pallas as pl
from jax.experimental.pallas import tpu as pltpu
from jax.experimental.pallas import tpu_sc as plsc

DURATION = 100.0
NWORKERS = 32


def _vgather(v, idx):
    return jax.lax.gather(
        v,
        idx[:, None],
        jax.lax.GatherDimensionNumbers(
            offset_dims=(), collapsed_slice_dims=(0,), start_index_map=(0,)
        ),
        (1,),
        mode=jax.lax.GatherScatterMode.PROMISE_IN_BOUNDS,
    )


def _sc_hist_body(off_hbm, out_hbm, buf, cnt_buf):
    nclass = 10
    nwords = off_hbm.shape[0] // NWORKERS
    nper = nwords // 16
    wid = lax.axis_index("s") * 2 + lax.axis_index("c")
    base = wid * nwords
    pltpu.sync_copy(off_hbm.at[pl.ds(base, nwords)], buf)

    lanes = lax.broadcasted_iota(jnp.int32, (16,), 0)
    real = lanes < nclass
    neg = jnp.full((16,), -3.4e38, jnp.float32)

    def neuron(i, cnt):
        v = jnp.where(real, buf[pl.ds(i * 16, 16)], neg)
        m = v
        for sh in (1, 2, 4, 8):
            m = jnp.maximum(m, _vgather(m, lanes ^ sh))
        a = jnp.where(v == m, lanes, 16)
        for sh in (1, 2, 4, 8):
            a = jnp.minimum(a, _vgather(a, lanes ^ sh))
        return cnt + jnp.where(lanes == a, 1.0, 0.0)

    cnt = lax.fori_loop(0, nper, neuron, jnp.zeros((16,), jnp.float32))
    cnt_buf[...] = cnt
    pltpu.sync_copy(cnt_buf, out_hbm.at[wid])


def _sc_histogram(offs_flat):
    nwords = offs_flat.shape[0] // NWORKERS
    mesh = plsc.VectorSubcoreMesh(
        core_axis_name="c", subcore_axis_name="s", num_cores=2
    )
    k = functools.partial(
        pl.kernel,
        mesh=mesh,
        out_type=jax.ShapeDtypeStruct((NWORKERS, 16), jnp.float32),
        scratch_types=[
            pltpu.VMEM((nwords,), jnp.float32),
            pltpu.VMEM((16,), jnp.float32),
        ],
    )(_sc_hist_body)
    return k(offs_flat)


def _tc_stream_body(x_ref, off_ref, out_ref, acc_ref, occ_ref):
    i = pl.program_id(0)
    nsteps = pl.num_programs(0)
    nclass = off_ref.shape[1]

    off = off_ref[...]
    norms = jnp.sum(jnp.abs(off), axis=1, keepdims=True)
    prop = off / jnp.maximum(norms, 1e-12)
    maxv = jnp.max(prop, axis=1, keepdims=True)
    iota = jax.lax.broadcasted_iota(jnp.int32, prop.shape, 1)
    amax = jnp.min(jnp.where(prop == maxv, iota, nclass), axis=1, keepdims=True)
    oh = (iota == amax).astype(jnp.bfloat16)
    assoc = jnp.where(iota == amax, prop, 0.0)

    @pl.when(i == 0)
    def _init():
        acc_ref[...] = jnp.zeros_like(acc_ref)
        occ_ref[...] = jnp.zeros_like(occ_ref)

    x = ((DURATION - x_ref[...]) * (1.0 / DURATION)).astype(jnp.bfloat16)
    acc_ref[...] += jnp.dot(
        x, assoc.astype(jnp.bfloat16), preferred_element_type=jnp.float32
    )
    ones = jnp.ones((8, oh.shape[0]), jnp.bfloat16)
    occ_ref[...] += jnp.dot(ones, oh, preferred_element_type=jnp.float32)

    @pl.when(i == nsteps - 1)
    def _fini():
        occ = jnp.maximum(occ_ref[0:1, :], 1.0)
        out_ref[...] = acc_ref[...] / occ


def _tc_stream(inputs, offsets):
    batch, nneuron = inputs.shape
    nclass = offsets.shape[1]
    blk_n = 4096
    grid = nneuron // blk_n
    return pl.pallas_call(
        _tc_stream_body,
        grid=(grid,),
        in_specs=[
            pl.BlockSpec((batch, blk_n), lambda i: (0, i)),
            pl.BlockSpec((blk_n, nclass), lambda i: (i, 0)),
        ],
        out_specs=pl.BlockSpec((batch, nclass), lambda i: (0, 0)),
        out_shape=jax.ShapeDtypeStruct((batch, nclass), jnp.float32),
        scratch_shapes=[
            pltpu.VMEM((batch, nclass), jnp.float32),
            pltpu.VMEM((8, nclass), jnp.float32),
        ],
        compiler_params=pltpu.CompilerParams(
            dimension_semantics=("arbitrary",),
        ),
    )(inputs, offsets)


def _tc_finalize_body(acc_ref, parts_ref, out_ref):
    nclass = acc_ref.shape[1]
    occ = jnp.sum(parts_ref[...], axis=0)[:nclass]
    out_ref[...] = acc_ref[...] / jnp.maximum(occ, 1.0)[None, :]


def _tc_finalize(acc, parts):
    return pl.pallas_call(
        _tc_finalize_body,
        out_shape=jax.ShapeDtypeStruct(acc.shape, jnp.float32),
    )(acc, parts)


def kernel(inputs, offsets):
    return _tc_stream(inputs, offsets)


# speedup vs baseline: 1.5021x; 1.0082x over previous
import functools

import jax
import jax.numpy as jnp
from jax import lax
from jax.experimental import pallas as pl
from jax.experimental.pallas import tpu as pltpu
from jax.experimental.pallas import tpu_sc as plsc

DURATION = 100.0
NWORKERS = 32


def _vgather(v, idx):
    return jax.lax.gather(
        v,
        idx[:, None],
        jax.lax.GatherDimensionNumbers(
            offset_dims=(), collapsed_slice_dims=(0,), start_index_map=(0,)
        ),
        (1,),
        mode=jax.lax.GatherScatterMode.PROMISE_IN_BOUNDS,
    )


def _sc_hist_body(off_hbm, out_hbm, buf, cnt_buf):
    nclass = 10
    nwords = off_hbm.shape[0] // NWORKERS
    nper = nwords // 16
    wid = lax.axis_index("s") * 2 + lax.axis_index("c")
    base = wid * nwords
    pltpu.sync_copy(off_hbm.at[pl.ds(base, nwords)], buf)

    lanes = lax.broadcasted_iota(jnp.int32, (16,), 0)
    real = lanes < nclass
    neg = jnp.full((16,), -3.4e38, jnp.float32)

    def neuron(i, cnt):
        v = jnp.where(real, buf[pl.ds(i * 16, 16)], neg)
        m = v
        for sh in (1, 2, 4, 8):
            m = jnp.maximum(m, _vgather(m, lanes ^ sh))
        a = jnp.where(v == m, lanes, 16)
        for sh in (1, 2, 4, 8):
            a = jnp.minimum(a, _vgather(a, lanes ^ sh))
        return cnt + jnp.where(lanes == a, 1.0, 0.0)

    cnt = lax.fori_loop(0, nper, neuron, jnp.zeros((16,), jnp.float32))
    cnt_buf[...] = cnt
    pltpu.sync_copy(cnt_buf, out_hbm.at[wid])


def _sc_histogram(offs_flat):
    nwords = offs_flat.shape[0] // NWORKERS
    mesh = plsc.VectorSubcoreMesh(
        core_axis_name="c", subcore_axis_name="s", num_cores=2
    )
    k = functools.partial(
        pl.kernel,
        mesh=mesh,
        out_type=jax.ShapeDtypeStruct((NWORKERS, 16), jnp.float32),
        scratch_types=[
            pltpu.VMEM((nwords,), jnp.float32),
            pltpu.VMEM((16,), jnp.float32),
        ],
    )(_sc_hist_body)
    return k(offs_flat)


def _tc_stream_body(x_ref, off_ref, out_ref):
    i = pl.program_id(0)
    nclass = off_ref.shape[1]

    off = off_ref[...]
    norms = jnp.sum(jnp.abs(off), axis=1, keepdims=True)
    prop = off / jnp.maximum(norms, 1e-12)
    maxv = jnp.max(prop, axis=1, keepdims=True)
    iota = jax.lax.broadcasted_iota(jnp.int32, prop.shape, 1)
    amax = jnp.min(jnp.where(prop == maxv, iota, nclass), axis=1, keepdims=True)
    assoc = jnp.where(iota == amax, prop, 0.0)

    @pl.when(i == 0)
    def _init():
        out_ref[...] = jnp.zeros_like(out_ref)

    x = ((DURATION - x_ref[...]) * (1.0 / DURATION)).astype(jnp.bfloat16)
    out_ref[...] += jnp.dot(
        x, assoc.astype(jnp.bfloat16), preferred_element_type=jnp.float32
    )


def _tc_stream(inputs, offsets):
    batch, nneuron = inputs.shape
    nclass = offsets.shape[1]
    blk_n = 4096
    grid = nneuron // blk_n
    return pl.pallas_call(
        _tc_stream_body,
        grid=(grid,),
        in_specs=[
            pl.BlockSpec((batch, blk_n), lambda i: (0, i)),
            pl.BlockSpec((blk_n, nclass), lambda i: (i, 0)),
        ],
        out_specs=pl.BlockSpec((batch, nclass), lambda i: (0, 0)),
        out_shape=jax.ShapeDtypeStruct((batch, nclass), jnp.float32),
        compiler_params=pltpu.CompilerParams(
            dimension_semantics=("arbitrary",),
        ),
    )(inputs, offsets)


def _tc_finalize_body(acc_ref, parts_ref, out_ref):
    nclass = acc_ref.shape[1]
    occ = jnp.sum(parts_ref[...], axis=0)[:nclass]
    out_ref[...] = acc_ref[...] / jnp.maximum(occ, 1.0)[None, :]


def _tc_finalize(acc, parts):
    return pl.pallas_call(
        _tc_finalize_body,
        out_shape=jax.ShapeDtypeStruct(acc.shape, jnp.float32),
    )(acc, parts)


def kernel(inputs, offsets):
    return _tc_stream(inputs, offsets)


# speedup vs baseline: 1.5938x; 1.0611x over previous
import jax
import jax.numpy as jnp
from jax.experimental import pallas as pl
from jax.experimental.pallas import tpu as pltpu

DURATION = 100.0


def _body(x_ref, offp_ref, out_ref, acc_ref, occ_ref):
    i = pl.program_id(0)
    nsteps = pl.num_programs(0)
    nclass = out_ref.shape[1]

    blk = offp_ref[...]
    off = jnp.concatenate(
        [blk[:, 16 * j : 16 * (j + 1)] for j in range(8)], axis=0
    )

    norms = jnp.sum(jnp.abs(off), axis=1, keepdims=True)
    prop = off / jnp.maximum(norms, 1e-12)
    maxv = jnp.max(prop, axis=1, keepdims=True)
    iota = jax.lax.broadcasted_iota(jnp.int32, prop.shape, 1)
    amax = jnp.min(jnp.where(prop == maxv, iota, 16), axis=1, keepdims=True)
    oh16 = iota == amax
    assoc = jnp.where(oh16, prop, 0.0)

    @pl.when(i == 0)
    def _init():
        acc_ref[...] = jnp.zeros_like(acc_ref)
        occ_ref[...] = jnp.zeros_like(occ_ref)

    x = ((DURATION - x_ref[...]) * (1.0 / DURATION)).astype(jnp.bfloat16)
    acc_ref[...] += jnp.dot(
        x, assoc.astype(jnp.bfloat16), preferred_element_type=jnp.float32
    )
    ones = jnp.ones((8, oh16.shape[0]), jnp.bfloat16)
    occ_ref[...] += jnp.dot(
        ones, oh16.astype(jnp.bfloat16), preferred_element_type=jnp.float32
    )

    @pl.when(i == nsteps - 1)
    def _fini():
        occ = jnp.maximum(occ_ref[0:1, :nclass], 1.0)
        out_ref[...] = acc_ref[:, :nclass] / occ


def kernel(inputs, offsets):
    batch, nneuron = inputs.shape
    nclass = offsets.shape[1]
    blk_n = 4096
    grid = nneuron // blk_n

    offs16 = jnp.pad(offsets, ((0, 0), (0, 16 - nclass)))
    offp = (
        offs16.reshape(grid, 8, 512, 16)
        .transpose(2, 0, 1, 3)
        .reshape(512, 8 * 16 * grid)
    )

    return pl.pallas_call(
        _body,
        grid=(grid,),
        in_specs=[
            pl.BlockSpec((batch, blk_n), lambda i: (0, i)),
            pl.BlockSpec((512, 128), lambda i: (0, i)),
        ],
        out_specs=pl.BlockSpec((batch, nclass), lambda i: (0, 0)),
        out_shape=jax.ShapeDtypeStruct((batch, nclass), jnp.float32),
        scratch_shapes=[
            pltpu.VMEM((batch, 16), jnp.float32),
            pltpu.VMEM((8, 16), jnp.float32),
        ],
        compiler_params=pltpu.CompilerParams(
            dimension_semantics=("arbitrary",),
        ),
    )(inputs, offp)
